# baseline (speedup 1.0000x reference)
import functools

import jax
import jax.numpy as jnp
from jax import lax
from jax.experimental import pallas as pl
from jax.experimental.pallas import tpu as pltpu
from jax.experimental.pallas import tpu_sc as plsc

NUM_EMBEDDINGS = 100000
EMBED_DIM = 64
BATCH = 4096
N_FIELDS = 26

NC = 2
NS = 16
NW = NC * NS

PAD_V = 100096
D_PER_W = EMBED_DIM // NW
NIDX = 3
GRP = BATCH // 16
UNROLL = 16


@functools.partial(
    pl.kernel,
    out_type=jax.ShapeDtypeStruct((N_FIELDS * EMBED_DIM, BATCH), jnp.float32),
    mesh=plsc.VectorSubcoreMesh(core_axis_name="c", subcore_axis_name="s"),
    compiler_params=pltpu.CompilerParams(use_tc_tiling_on_sc=False, needs_layout_passes=False),
    scratch_types=[
        pltpu.VMEM((PAD_V,), jnp.float32),
        [pltpu.VMEM((BATCH,), jnp.int32) for _ in range(NIDX)],
        pltpu.VMEM((BATCH,), jnp.float32),
        pltpu.VMEM((BATCH,), jnp.float32),
        [pltpu.SemaphoreType.DMA for _ in range(NIDX)],
        pltpu.SemaphoreType.DMA,
        pltpu.SemaphoreType.DMA,
        pltpu.SemaphoreType.DMA,
    ],
)
def _sc_gather(idx_hbm, wt_hbm, out_hbm, spm, idx_vs, stage0, stage1,
               isems, ssem0, ssem1, wsem):
    wid = lax.axis_index("s") * NC + lax.axis_index("c")
    stages = (stage0, stage1)
    ssems = (ssem0, ssem1)

    def idx_row(f):
        return idx_hbm.at[pl.ds(f * BATCH, BATCH)]

    def start_idx(f, slot):
        pltpu.async_copy(idx_row(f), idx_vs[slot], isems[slot])

    def wait_idx(f, slot):
        pltpu.make_async_copy(idx_row(f), idx_vs[slot], isems[slot]).wait()

    nseq = D_PER_W * N_FIELDS
    for di in range(D_PER_W):
        d = wid * D_PER_W + di
        pltpu.async_copy(wt_hbm.at[d], spm, wsem)
        if di == 0:
            for s in range(NIDX):
                start_idx(s % N_FIELDS, s % NIDX)
        pltpu.make_async_copy(wt_hbm.at[d], spm, wsem).wait()
        for f in range(N_FIELDS):
            seq = di * N_FIELDS + f
            slot = seq % NIDX
            st = f % 2
            wait_idx(f, slot)
            if f >= 2:
                pltpu.make_async_copy(
                    stages[st], out_hbm.at[(f - 2) * EMBED_DIM + d], ssems[st]
                ).wait()
            idxv = idx_vs[slot]
            stage = stages[st]

            @plsc.parallel_loop(0, BATCH, step=16, unroll=UNROLL)
            def body(b):
                v = plsc.load_gather(spm, [idxv[pl.ds(b, 16)]])
                stage[pl.ds(b, 16)] = v
            if seq + NIDX < nseq:
                start_idx((seq + NIDX) % N_FIELDS, slot)
            pltpu.async_copy(stage, out_hbm.at[f * EMBED_DIM + d], ssems[st])
        for f in (N_FIELDS - 2, N_FIELDS - 1):
            pltpu.make_async_copy(
                stages[f % 2], out_hbm.at[f * EMBED_DIM + d], ssems[f % 2]
            ).wait()


def kernel(input_tensor, weight):
    idx_t = input_tensor.T.reshape(N_FIELDS * BATCH)
    wt = jnp.pad(weight.T, ((0, 0), (0, PAD_V - NUM_EMBEDDINGS - 1)))
    out_t = _sc_gather(idx_t, wt)
    return out_t.reshape(N_FIELDS, EMBED_DIM, BATCH).transpose(2, 0, 1)


# speedup vs baseline: 1.2814x; 1.2814x over previous
import functools

import jax
import jax.numpy as jnp
from jax import lax
from jax.experimental import pallas as pl
from jax.experimental.pallas import tpu as pltpu
from jax.experimental.pallas import tpu_sc as plsc

NUM_EMBEDDINGS = 100000
EMBED_DIM = 64
BATCH = 4096
N_FIELDS = 26

NC = 2
NS = 16
NW = NC * NS

PAD_V = 100096
D_PER_W = EMBED_DIM // NW
NIDX = 3
GRP = BATCH // 16
UNROLL = 8


@functools.partial(
    pl.kernel,
    out_type=jax.ShapeDtypeStruct((N_FIELDS, 8, 32, 8, 128), jnp.float32),
    mesh=plsc.VectorSubcoreMesh(core_axis_name="c", subcore_axis_name="s"),
    compiler_params=pltpu.CompilerParams(use_tc_tiling_on_sc=False, needs_layout_passes=False),
    scratch_types=[
        pltpu.VMEM((PAD_V,), jnp.float32),
        [pltpu.VMEM((BATCH,), jnp.int32) for _ in range(NIDX)],
        pltpu.VMEM((32, 128), jnp.float32),
        pltpu.VMEM((32, 128), jnp.float32),
        [pltpu.SemaphoreType.DMA for _ in range(NIDX)],
        pltpu.SemaphoreType.DMA,
        pltpu.SemaphoreType.DMA,
        pltpu.SemaphoreType.DMA,
    ],
)
def _sc_gather(idx_hbm, wt_hbm, out_hbm, spm, idx_vs, stage0, stage1,
               isems, ssem0, ssem1, wsem):
    wid = lax.axis_index("s") * NC + lax.axis_index("c")
    stages = (stage0, stage1)
    ssems = (ssem0, ssem1)

    def idx_row(f):
        return idx_hbm.at[pl.ds(f * BATCH, BATCH)]

    def start_idx(f, slot):
        pltpu.async_copy(idx_row(f), idx_vs[slot], isems[slot])

    def wait_idx(f, slot):
        pltpu.make_async_copy(idx_row(f), idx_vs[slot], isems[slot]).wait()

    nseq = D_PER_W * N_FIELDS
    for di in range(D_PER_W):
        d = wid * D_PER_W + di
        pltpu.async_copy(wt_hbm.at[d], spm, wsem)
        if di == 0:
            for s in range(NIDX):
                start_idx(s % N_FIELDS, s % NIDX)
        pltpu.make_async_copy(wt_hbm.at[d], spm, wsem).wait()
        for f in range(N_FIELDS):
            seq = di * N_FIELDS + f
            slot = seq % NIDX
            st = f % 2
            wait_idx(f, slot)
            if f >= 2:
                pltpu.make_async_copy(
                    stages[st], out_hbm.at[f - 2, d // 8, :, d % 8, :], ssems[st]
                ).wait()
            idxv = idx_vs[slot]
            stage = stages[st]

            @plsc.parallel_loop(0, BATCH, step=16, unroll=UNROLL)
            def body(b):
                v = plsc.load_gather(spm, [idxv[pl.ds(b, 16)]])
                stage[b // 128, pl.ds(b % 128, 16)] = v
            if seq + NIDX < nseq:
                start_idx((seq + NIDX) % N_FIELDS, slot)
            pltpu.async_copy(stage, out_hbm.at[f, d // 8, :, d % 8, :], ssems[st])
        for f in (N_FIELDS - 2, N_FIELDS - 1):
            pltpu.make_async_copy(
                stages[f % 2], out_hbm.at[f, d // 8, :, d % 8, :], ssems[f % 2]
            ).wait()


def kernel(input_tensor, weight):
    idx_t = input_tensor.T.reshape(N_FIELDS * BATCH)
    wt = jnp.pad(weight.T, ((0, 0), (0, PAD_V - NUM_EMBEDDINGS - 1)))
    out_t = _sc_gather(idx_t, wt)
    return out_t.transpose(2, 4, 0, 1, 3).reshape(BATCH, N_FIELDS, EMBED_DIM)


# speedup vs baseline: 1.5285x; 1.1928x over previous
import functools

import jax
import jax.numpy as jnp
from jax import lax
from jax.experimental import pallas as pl
from jax.experimental.pallas import tpu as pltpu
from jax.experimental.pallas import tpu_sc as plsc

NUM_EMBEDDINGS = 100000
EMBED_DIM = 64
BATCH = 4096
N_FIELDS = 26

NC = 2
NS = 16
NW = NC * NS

PAD_V = 100096
D_PER_W = EMBED_DIM // NW
NIDX = 3
GRP = BATCH // 16
UNROLL = 8


@functools.partial(
    pl.kernel,
    out_type=jax.ShapeDtypeStruct((N_FIELDS, 8, 32, 8, 128), jnp.float32),
    mesh=plsc.VectorSubcoreMesh(core_axis_name="c", subcore_axis_name="s"),
    compiler_params=pltpu.CompilerParams(use_tc_tiling_on_sc=False, needs_layout_passes=False),
    scratch_types=[
        pltpu.VMEM((PAD_V // 128, 128), jnp.float32),
        [pltpu.VMEM((BATCH,), jnp.int32) for _ in range(NIDX)],
        pltpu.VMEM((32, 128), jnp.float32),
        pltpu.VMEM((32, 128), jnp.float32),
        [pltpu.SemaphoreType.DMA for _ in range(NIDX)],
        pltpu.SemaphoreType.DMA,
        pltpu.SemaphoreType.DMA,
        pltpu.SemaphoreType.DMA,
    ],
)
def _sc_gather(idx_hbm, wt_hbm, out_hbm, spm, idx_vs, stage0, stage1,
               isems, ssem0, ssem1, wsem):
    wid = lax.axis_index("s") * NC + lax.axis_index("c")
    stages = (stage0, stage1)
    ssems = (ssem0, ssem1)

    def idx_row(f):
        return idx_hbm.at[pl.ds(f * BATCH, BATCH)]

    def start_idx(f, slot):
        pltpu.async_copy(idx_row(f), idx_vs[slot], isems[slot])

    def wait_idx(f, slot):
        pltpu.make_async_copy(idx_row(f), idx_vs[slot], isems[slot]).wait()

    nseq = D_PER_W * N_FIELDS
    for di in range(D_PER_W):
        d = wid * D_PER_W + di
        pltpu.async_copy(wt_hbm.at[d // 8, :, d % 8, :], spm, wsem)
        if di == 0:
            for s in range(NIDX):
                start_idx(s % N_FIELDS, s % NIDX)
        pltpu.make_async_copy(wt_hbm.at[d // 8, :, d % 8, :], spm, wsem).wait()
        for f in range(N_FIELDS):
            seq = di * N_FIELDS + f
            slot = seq % NIDX
            st = f % 2
            wait_idx(f, slot)
            if f >= 2:
                pltpu.make_async_copy(
                    stages[st], out_hbm.at[f - 2, d // 8, :, d % 8, :], ssems[st]
                ).wait()
            idxv = idx_vs[slot]
            stage = stages[st]

            @plsc.parallel_loop(0, BATCH, step=16, unroll=UNROLL)
            def body(b):
                iv = idxv[pl.ds(b, 16)]
                v = plsc.load_gather(spm, [iv >> 7, iv & 127])
                stage[b // 128, pl.ds(b % 128, 16)] = v
            if seq + NIDX < nseq:
                start_idx((seq + NIDX) % N_FIELDS, slot)
            pltpu.async_copy(stage, out_hbm.at[f, d // 8, :, d % 8, :], ssems[st])
        for f in (N_FIELDS - 2, N_FIELDS - 1):
            pltpu.make_async_copy(
                stages[f % 2], out_hbm.at[f, d // 8, :, d % 8, :], ssems[f % 2]
            ).wait()


def kernel(input_tensor, weight):
    idx_t = input_tensor.T.reshape(N_FIELDS * BATCH)
    wt = jnp.pad(weight.T, ((0, 0), (0, PAD_V - NUM_EMBEDDINGS - 1)))
    wt4 = wt.reshape(8, 8, PAD_V // 128, 128).transpose(0, 2, 1, 3)
    out_t = _sc_gather(idx_t, wt4)
    return out_t.transpose(2, 4, 0, 1, 3).reshape(BATCH, N_FIELDS, EMBED_DIM)
